# Initial kernel scaffold; baseline (speedup 1.0000x reference)
import jax
import jax.numpy as jnp
from jax.experimental import pallas as pl


def kernel(features, layer_idx, layer_embeddings):
    raise NotImplementedError("write your pallas kernel here")


# speedup vs baseline: 1.0028x; 1.0028x over previous
import jax
import jax.numpy as jnp
from jax.experimental import pallas as pl
from jax.experimental.pallas import tpu as pltpu

_BLK = 512


def _add_body(idx_ref, emb_ref, x_ref, o_ref):
    row = emb_ref[pl.ds(idx_ref[0], 1), :]
    o_ref[...] = x_ref[...] + row


def kernel(features, layer_idx, layer_embeddings):
    B, S, D = features.shape
    M = B * S
    x2d = features.reshape(M, D)
    idx_arr = jnp.asarray(layer_idx, dtype=jnp.int32).reshape(1)
    grid_spec = pltpu.PrefetchScalarGridSpec(
        num_scalar_prefetch=1,
        grid=(M // _BLK,),
        in_specs=[
            pl.BlockSpec(layer_embeddings.shape, lambda i, idx: (0, 0)),
            pl.BlockSpec((_BLK, D), lambda i, idx: (i, 0)),
        ],
        out_specs=pl.BlockSpec((_BLK, D), lambda i, idx: (i, 0)),
    )
    out = pl.pallas_call(
        _add_body,
        grid_spec=grid_spec,
        out_shape=jax.ShapeDtypeStruct((M, D), jnp.float32),
        compiler_params=pltpu.CompilerParams(
            dimension_semantics=("arbitrary",),
        ),
    )(idx_arr, layer_embeddings, x2d)
    return out.reshape(B, S, D)
